# Initial kernel scaffold; baseline (speedup 1.0000x reference)
import jax
import jax.numpy as jnp
from jax.experimental import pallas as pl


def kernel(x, edge_index, edge_attr, W, b):
    raise NotImplementedError("write your pallas kernel here")


# speedup vs baseline: 4.1744x; 4.1744x over previous
import jax
import jax.numpy as jnp
from jax import lax
from jax.experimental import pallas as pl
from jax.experimental.pallas import tpu as pltpu
from jax.experimental.pallas import tpu_sc as plsc

N_NODES = 10000
N_EDGES = 320000
D_FEAT = 128
D_EDGE = 16
D_OUT = 128

NC = 2
NS = 16
L = 16
NW = NC * NS
EPW = N_EDGES // NW
CHUNK = 80
NCHUNK = EPW // CHUNK
GROUPS = CHUNK // L
SUB_ROWS = 624
TAIL_ROWS = N_NODES - NS * SUB_ROWS


def _rsqrt16(t):
    i = plsc.bitcast(t, jnp.int32)
    i = jnp.int32(0x5F3759DF) - (i >> 1)
    y = plsc.bitcast(i, jnp.float32)
    for _ in range(3):
        y = y * (jnp.float32(1.5) - jnp.float32(0.5) * t * y * y)
    return y


def _sc_kernel_body(x_hbm, row_hbm, col_hbm, ea_hbm, sqn_hbm,
                    outx_hbm, oute_hbm,
                    sqn_v, rowi, coli, eav, xg, ebuf,
                    accx, acce, sem):
    c = lax.axis_index("c")
    s = lax.axis_index("s")
    wid = s * NC + c
    wbase = wid * EPW

    pltpu.sync_copy(sqn_hbm, sqn_v)

    z16 = jnp.zeros((16,), jnp.float32)

    def zero_bufs(i, _):
        for k in range(8):
            xg[i, pl.ds(16 * k, 16)] = z16
        for k in range(2):
            ebuf[i, pl.ds(16 * k, 16)] = z16
        return 0

    lax.fori_loop(0, CHUNK, zero_bufs, 0)

    ncopies = jnp.where(s == NS - 1, 5, 8)

    def zero_acc(i, _):
        abase = s * (8 * CHUNK) + i * CHUNK
        pltpu.sync_copy(xg, accx.at[pl.ds(abase, CHUNK)])
        pltpu.sync_copy(ebuf, acce.at[pl.ds(abase, CHUNK)])
        return 0

    lax.fori_loop(0, ncopies, zero_acc, 0)
    plsc.subcore_barrier()

    e1 = jnp.where(lax.iota(jnp.int32, 16) == 0, jnp.float32(1.0), jnp.float32(0.0))

    def set_ones(i, _):
        ebuf[i, pl.ds(16, 16)] = e1
        return 0

    lax.fori_loop(0, CHUNK, set_ones, 0)

    def chunk_body(ci, _):
        base = wbase + ci * CHUNK
        pltpu.sync_copy(row_hbm.at[pl.ds(base, CHUNK)], rowi)
        pltpu.sync_copy(col_hbm.at[pl.ds(base, CHUNK)], coli)
        pltpu.sync_copy(ea_hbm.at[pl.ds(base, CHUNK)], eav)
        pltpu.async_copy(x_hbm.at[rowi], xg, sem).wait()

        for g in range(GROUPS):
            r16 = rowi[pl.ds(g * L, L)]
            t = plsc.load_gather(sqn_v, [r16])
            rids = lax.iota(jnp.int32, 16) + jnp.int32(g * L)
            for j in range(L):
                cv = plsc.load_gather(eav, [rids, jnp.full((16,), j, jnp.int32)])
                t = t + cv * cv
            y = _rsqrt16(t)
            for j in range(L):
                e = g * L + j
                w = y[j]
                for k in range(8):
                    xg[e, pl.ds(16 * k, 16)] = xg[e, pl.ds(16 * k, 16)] * w
                ebuf[e, pl.ds(0, 16)] = eav[e] * w

        pltpu.sync_copy(xg, accx.at[coli], add=True)
        pltpu.sync_copy(ebuf, acce.at[coli], add=True)
        return 0

    lax.fori_loop(0, NCHUNK, chunk_body, 0)
    plsc.subcore_barrier()

    sub_base = s * SUB_ROWS
    pltpu.sync_copy(accx.at[pl.ds(sub_base, SUB_ROWS)],
                    outx_hbm.at[c, pl.ds(sub_base, SUB_ROWS)])
    pltpu.sync_copy(acce.at[pl.ds(sub_base, SUB_ROWS)],
                    oute_hbm.at[c, pl.ds(sub_base, SUB_ROWS)])

    @pl.when(s == NS - 1)
    def _copy_tail():
        tail = NS * SUB_ROWS
        pltpu.sync_copy(accx.at[pl.ds(tail, TAIL_ROWS)],
                        outx_hbm.at[c, pl.ds(tail, TAIL_ROWS)])
        pltpu.sync_copy(acce.at[pl.ds(tail, TAIL_ROWS)],
                        oute_hbm.at[c, pl.ds(tail, TAIL_ROWS)])


def _make_sc_kernel():
    mesh = plsc.VectorSubcoreMesh(core_axis_name="c", subcore_axis_name="s")
    return pl.kernel(
        _sc_kernel_body,
        out_type=[
            jax.ShapeDtypeStruct((NC, N_NODES, D_FEAT), jnp.float32),
            jax.ShapeDtypeStruct((NC, N_NODES, 32), jnp.float32),
        ],
        mesh=mesh,
        scratch_types=[
            pltpu.VMEM((N_NODES,), jnp.float32),
            pltpu.VMEM((CHUNK,), jnp.int32),
            pltpu.VMEM((CHUNK,), jnp.int32),
            pltpu.VMEM((CHUNK, D_EDGE), jnp.float32),
            pltpu.VMEM((CHUNK, D_FEAT), jnp.float32),
            pltpu.VMEM((CHUNK, 32), jnp.float32),
            pltpu.VMEM_SHARED((N_NODES, D_FEAT), jnp.float32),
            pltpu.VMEM_SHARED((N_NODES, 32), jnp.float32),
            pltpu.SemaphoreType.DMA,
        ],
        compiler_params=pltpu.CompilerParams(
            needs_layout_passes=False, use_tc_tiling_on_sc=False),
    )


def _sqn_tc_body(x_ref, o_ref):
    x = x_ref[...]
    o_ref[...] = jnp.sum(x * x, axis=1, keepdims=True)


def _finish_tc_body(accx_ref, acce_ref, wxt_ref, wet_ref, b_ref, o_ref):
    sx = accx_ref[0] + accx_ref[1]
    se = acce_ref[0] + acce_ref[1]
    cnt = se[:, 16:17]
    denom = jnp.maximum(cnt, 1.0)
    mx = sx / denom
    me = se[:, 0:16] / denom
    o = (jnp.dot(mx, wxt_ref[...], preferred_element_type=jnp.float32)
         + jnp.dot(me, wet_ref[...], preferred_element_type=jnp.float32)
         + b_ref[...])
    nrm = jnp.sqrt(jnp.sum(o * o, axis=1, keepdims=True))
    o_ref[...] = o / jnp.maximum(nrm, 1e-12)


@jax.jit
def kernel(x, edge_index, edge_attr, W, b):
    row = edge_index[0].astype(jnp.int32)
    col = edge_index[1].astype(jnp.int32)

    rblk = 2000
    sqn2 = pl.pallas_call(
        _sqn_tc_body,
        grid=(N_NODES // rblk,),
        in_specs=[pl.BlockSpec((rblk, D_FEAT), lambda i: (i, 0))],
        out_specs=pl.BlockSpec((rblk, 1), lambda i: (i, 0)),
        out_shape=jax.ShapeDtypeStruct((N_NODES, 1), jnp.float32),
    )(x)
    sqn = sqn2.reshape((N_NODES,))

    accx, acce = _make_sc_kernel()(x, row, col, edge_attr, sqn)

    wxt = W[:, :D_FEAT].T
    wet = W[:, D_FEAT:].T
    b2 = b.reshape((1, D_OUT))
    out = pl.pallas_call(
        _finish_tc_body,
        grid=(N_NODES // rblk,),
        in_specs=[
            pl.BlockSpec((NC, rblk, D_FEAT), lambda i: (0, i, 0)),
            pl.BlockSpec((NC, rblk, 32), lambda i: (0, i, 0)),
            pl.BlockSpec((D_FEAT, D_OUT), lambda i: (0, 0)),
            pl.BlockSpec((D_EDGE, D_OUT), lambda i: (0, 0)),
            pl.BlockSpec((1, D_OUT), lambda i: (0, 0)),
        ],
        out_specs=pl.BlockSpec((rblk, D_OUT), lambda i: (i, 0)),
        out_shape=jax.ShapeDtypeStruct((N_NODES, D_OUT), jnp.float32),
    )(accx, acce, wxt, wet, b2)
    return out


# speedup vs baseline: 6.0189x; 1.4418x over previous
import jax
import jax.numpy as jnp
from jax import lax
from jax.experimental import pallas as pl
from jax.experimental.pallas import tpu as pltpu
from jax.experimental.pallas import tpu_sc as plsc

N_NODES = 10000
N_EDGES = 320000
D_FEAT = 128
D_EDGE = 16
D_OUT = 128

NC = 2
NS = 16
L = 16
NW = NC * NS
EPW = N_EDGES // NW
CHUNK = 80
NCHUNK = EPW // CHUNK
GROUPS = CHUNK // L
SUB_ROWS = 624
TAIL_ROWS = N_NODES - NS * SUB_ROWS


def _rsqrt16(t):
    i = plsc.bitcast(t, jnp.int32)
    i = jnp.int32(0x5F3759DF) - (i >> 1)
    y = plsc.bitcast(i, jnp.float32)
    for _ in range(3):
        y = y * (jnp.float32(1.5) - jnp.float32(0.5) * t * y * y)
    return y


def _sc_kernel_body(x_hbm, row_hbm, col_hbm, ea_hbm, sqn_hbm,
                    outx_hbm, oute_hbm,
                    rowi, coli, colis, eav, xg, ebuf, sqc,
                    accx, acce,
                    semi, semg, sems):
    c = lax.axis_index("c")
    s = lax.axis_index("s")
    wid = s * NC + c
    wbase = wid * EPW

    z16 = jnp.zeros((16,), jnp.float32)

    def zero_bufs(i, _):
        for k in range(8):
            xg[0][i, pl.ds(16 * k, 16)] = z16
        for k in range(2):
            ebuf[0][i, pl.ds(16 * k, 16)] = z16
        return 0

    lax.fori_loop(0, CHUNK, zero_bufs, 0)

    ncopies = jnp.where(s == NS - 1, 5, 8)

    def zero_acc(i, _):
        abase = s * (8 * CHUNK) + i * CHUNK
        pltpu.sync_copy(xg[0], accx.at[pl.ds(abase, CHUNK)])
        pltpu.sync_copy(ebuf[0], acce.at[pl.ds(abase, CHUNK)])
        return 0

    lax.fori_loop(0, ncopies, zero_acc, 0)
    plsc.subcore_barrier()

    e1 = jnp.where(lax.iota(jnp.int32, 16) == 0, jnp.float32(1.0), jnp.float32(0.0))

    def set_ones(i, _):
        ebuf[0][i, pl.ds(16, 16)] = e1
        ebuf[1][i, pl.ds(16, 16)] = e1
        return 0

    lax.fori_loop(0, CHUNK, set_ones, 0)

    def issue_idx(b, ci):
        base = wbase + ci * CHUNK
        pltpu.async_copy(row_hbm.at[pl.ds(base, CHUNK)], rowi[b], semi[b])
        pltpu.async_copy(col_hbm.at[pl.ds(base, CHUNK)], coli[b], semi[b])
        pltpu.async_copy(ea_hbm.at[pl.ds(base, CHUNK)], eav[b], semi[b])

    def wait_idx(b):
        pltpu.make_async_copy(row_hbm.at[pl.ds(0, CHUNK)], rowi[b], semi[b]).wait()
        pltpu.make_async_copy(col_hbm.at[pl.ds(0, CHUNK)], coli[b], semi[b]).wait()
        pltpu.make_async_copy(ea_hbm.at[pl.ds(0, CHUNK)], eav[b], semi[b]).wait()

    def issue_gather(b):
        pltpu.async_copy(x_hbm.at[rowi[b]], xg[b], semg[b])
        pltpu.async_copy(sqn_hbm.at[rowi[b]], sqc[b], semg[b])

    def wait_gather(b):
        pltpu.make_async_copy(x_hbm.at[rowi[b]], xg[b], semg[b]).wait()
        pltpu.make_async_copy(sqn_hbm.at[rowi[b]], sqc[b], semg[b]).wait()

    def issue_scatter(b):
        pltpu.async_copy(xg[b], accx.at[colis[b]], sems[b], add=True)
        pltpu.async_copy(ebuf[b], acce.at[colis[b]], sems[b], add=True)

    def wait_scatter(b):
        pltpu.make_async_copy(xg[b], accx.at[colis[b]], sems[b]).wait()
        pltpu.make_async_copy(ebuf[b], acce.at[colis[b]], sems[b]).wait()

    def compute(b):
        for g in range(GROUPS):
            t = sqc[b][pl.ds(g * L, L)]
            rids = lax.iota(jnp.int32, 16) + jnp.int32(g * L)
            for j in range(L):
                cv = plsc.load_gather(eav[b], [rids, jnp.full((16,), j, jnp.int32)])
                t = t + cv * cv
            y = _rsqrt16(t)
            colis[b][pl.ds(g * L, L)] = coli[b][pl.ds(g * L, L)]
            for j in range(L):
                e = g * L + j
                w = y[j]
                for k in range(8):
                    xg[b][e, pl.ds(16 * k, 16)] = xg[b][e, pl.ds(16 * k, 16)] * w
                ebuf[b][e, pl.ds(0, 16)] = eav[b][e] * w

    NPAIR = NCHUNK // 2

    issue_idx(0, 0)
    issue_idx(1, 1)

    def pair_body(p, _):
        ca = 2 * p
        cb = 2 * p + 1

        @pl.when(p > 0)
        def _wa():
            wait_scatter(0)

        wait_idx(0)
        issue_gather(0)

        @pl.when(p > 0)
        def _wb():
            wait_scatter(1)

        wait_idx(1)
        issue_gather(1)

        wait_gather(0)
        compute(0)
        issue_scatter(0)
        issue_idx(0, ca + 2)

        wait_gather(1)
        compute(1)
        issue_scatter(1)

        @pl.when(p < NPAIR - 1)
        def _pf():
            issue_idx(1, cb + 2)

        return 0

    lax.fori_loop(0, NPAIR, pair_body, 0)

    wait_scatter(0)
    wait_idx(0)
    issue_gather(0)
    wait_scatter(1)
    wait_gather(0)
    compute(0)
    issue_scatter(0)
    wait_scatter(0)

    plsc.subcore_barrier()

    sub_base = s * SUB_ROWS
    pltpu.sync_copy(accx.at[pl.ds(sub_base, SUB_ROWS)],
                    outx_hbm.at[c, pl.ds(sub_base, SUB_ROWS)])
    pltpu.sync_copy(acce.at[pl.ds(sub_base, SUB_ROWS)],
                    oute_hbm.at[c, pl.ds(sub_base, SUB_ROWS)])

    @pl.when(s == NS - 1)
    def _copy_tail():
        tail = NS * SUB_ROWS
        pltpu.sync_copy(accx.at[pl.ds(tail, TAIL_ROWS)],
                        outx_hbm.at[c, pl.ds(tail, TAIL_ROWS)])
        pltpu.sync_copy(acce.at[pl.ds(tail, TAIL_ROWS)],
                        oute_hbm.at[c, pl.ds(tail, TAIL_ROWS)])


def _make_sc_kernel():
    mesh = plsc.VectorSubcoreMesh(core_axis_name="c", subcore_axis_name="s")
    return pl.kernel(
        _sc_kernel_body,
        out_type=[
            jax.ShapeDtypeStruct((NC, N_NODES, D_FEAT), jnp.float32),
            jax.ShapeDtypeStruct((NC, N_NODES, 32), jnp.float32),
        ],
        mesh=mesh,
        scratch_types=[
            (pltpu.VMEM((CHUNK,), jnp.int32),) * 2,
            (pltpu.VMEM((CHUNK,), jnp.int32),) * 2,
            (pltpu.VMEM((CHUNK,), jnp.int32),) * 2,
            (pltpu.VMEM((CHUNK, D_EDGE), jnp.float32),) * 2,
            (pltpu.VMEM((CHUNK, D_FEAT), jnp.float32),) * 2,
            (pltpu.VMEM((CHUNK, 32), jnp.float32),) * 2,
            (pltpu.VMEM((CHUNK,), jnp.float32),) * 2,
            pltpu.VMEM_SHARED((N_NODES, D_FEAT), jnp.float32),
            pltpu.VMEM_SHARED((N_NODES, 32), jnp.float32),
            (pltpu.SemaphoreType.DMA,) * 2,
            (pltpu.SemaphoreType.DMA,) * 2,
            (pltpu.SemaphoreType.DMA,) * 2,
        ],
        compiler_params=pltpu.CompilerParams(
            needs_layout_passes=False, use_tc_tiling_on_sc=False),
    )


def _sqn_tc_body(x_ref, o_ref):
    x = x_ref[...]
    o_ref[...] = jnp.sum(x * x, axis=1, keepdims=True)


def _finish_tc_body(accx_ref, acce_ref, wxt_ref, wet_ref, b_ref, o_ref):
    sx = accx_ref[0] + accx_ref[1]
    se = acce_ref[0] + acce_ref[1]
    cnt = se[:, 16:17]
    denom = jnp.maximum(cnt, 1.0)
    mx = sx / denom
    me = se[:, 0:16] / denom
    o = (jnp.dot(mx, wxt_ref[...], preferred_element_type=jnp.float32)
         + jnp.dot(me, wet_ref[...], preferred_element_type=jnp.float32)
         + b_ref[...])
    nrm = jnp.sqrt(jnp.sum(o * o, axis=1, keepdims=True))
    o_ref[...] = o / jnp.maximum(nrm, 1e-12)


@jax.jit
def kernel(x, edge_index, edge_attr, W, b):
    row = edge_index[0].astype(jnp.int32)
    col = edge_index[1].astype(jnp.int32)

    rblk = 2000
    sqn2 = pl.pallas_call(
        _sqn_tc_body,
        grid=(N_NODES // rblk,),
        in_specs=[pl.BlockSpec((rblk, D_FEAT), lambda i: (i, 0))],
        out_specs=pl.BlockSpec((rblk, 1), lambda i: (i, 0)),
        out_shape=jax.ShapeDtypeStruct((N_NODES, 1), jnp.float32),
    )(x)
    sqn = sqn2.reshape((N_NODES,))

    accx, acce = _make_sc_kernel()(x, row, col, edge_attr, sqn)

    wxt = W[:, :D_FEAT].T
    wet = W[:, D_FEAT:].T
    b2 = b.reshape((1, D_OUT))
    out = pl.pallas_call(
        _finish_tc_body,
        grid=(N_NODES // rblk,),
        in_specs=[
            pl.BlockSpec((NC, rblk, D_FEAT), lambda i: (0, i, 0)),
            pl.BlockSpec((NC, rblk, 32), lambda i: (0, i, 0)),
            pl.BlockSpec((D_FEAT, D_OUT), lambda i: (0, 0)),
            pl.BlockSpec((D_EDGE, D_OUT), lambda i: (0, 0)),
            pl.BlockSpec((1, D_OUT), lambda i: (0, 0)),
        ],
        out_specs=pl.BlockSpec((rblk, D_OUT), lambda i: (i, 0)),
        out_shape=jax.ShapeDtypeStruct((N_NODES, D_OUT), jnp.float32),
    )(accx, acce, wxt, wet, b2)
    return out
